# baseline (speedup 1.0000x reference)
import jax
import jax.numpy as jnp
from jax.experimental import pallas as pl


_ROWS = 768
_COLS = 50176
_BM = 8


def _relu_block(x_ref, o_ref):
    o_ref[...] = jnp.maximum(x_ref[...], 0.0)


def kernel(x):
    x2 = x.reshape(_ROWS, _COLS)
    out = pl.pallas_call(
        _relu_block,
        grid=(_ROWS // _BM,),
        in_specs=[pl.BlockSpec((_BM, _COLS), lambda i: (i, 0))],
        out_specs=pl.BlockSpec((_BM, _COLS), lambda i: (i, 0)),
        out_shape=jax.ShapeDtypeStruct((_ROWS, _COLS), x.dtype),
    )(x2)
    return out.reshape(x.shape)


# speedup vs baseline: 1.0504x; 1.0504x over previous
import jax
import jax.numpy as jnp
from jax.experimental import pallas as pl
from jax.experimental.pallas import tpu as pltpu


_ROWS = 768
_COLS = 50176
_BM = 8
_N = _ROWS // _BM
_K = 8


def _relu_stream(x_hbm, o_hbm, in_buf, out_buf, in_sem, out_sem):
    def start_in(i, s):
        pltpu.make_async_copy(
            x_hbm.at[pl.ds(i * _BM, _BM), :], in_buf.at[s], in_sem.at[s]
        ).start()

    for s in range(_K):
        start_in(s, s)

    def body(i, _):
        s = jax.lax.rem(i, _K)
        pltpu.make_async_copy(
            x_hbm.at[pl.ds(i * _BM, _BM), :], in_buf.at[s], in_sem.at[s]
        ).wait()

        @pl.when(i >= _K)
        def _():
            pltpu.make_async_copy(
                out_buf.at[s], o_hbm.at[pl.ds((i - _K) * _BM, _BM), :],
                out_sem.at[s],
            ).wait()

        out_buf[s] = jnp.maximum(in_buf[s], 0.0)

        pltpu.make_async_copy(
            out_buf.at[s], o_hbm.at[pl.ds(i * _BM, _BM), :], out_sem.at[s]
        ).start()

        @pl.when(i + _K < _N)
        def _():
            start_in(i + _K, s)

        return ()

    jax.lax.fori_loop(0, _N, body, (), unroll=False)

    for j in range(_K):
        i = _N - _K + j
        s = i % _K
        pltpu.make_async_copy(
            out_buf.at[s], o_hbm.at[pl.ds(i * _BM, _BM), :], out_sem.at[s]
        ).wait()


def kernel(x):
    x2 = x.reshape(_ROWS, _COLS)
    out = pl.pallas_call(
        _relu_stream,
        in_specs=[pl.BlockSpec(memory_space=pl.ANY)],
        out_specs=pl.BlockSpec(memory_space=pl.ANY),
        out_shape=jax.ShapeDtypeStruct((_ROWS, _COLS), x.dtype),
        scratch_shapes=[
            pltpu.VMEM((_K, _BM, _COLS), jnp.float32),
            pltpu.VMEM((_K, _BM, _COLS), jnp.float32),
            pltpu.SemaphoreType.DMA((_K,)),
            pltpu.SemaphoreType.DMA((_K,)),
        ],
    )(x2)
    return out.reshape(x.shape)


# speedup vs baseline: 1.0516x; 1.0011x over previous
import jax
import jax.numpy as jnp
from jax.experimental import pallas as pl
from jax.experimental.pallas import tpu as pltpu


_ROWS = 768
_COLS = 50176
_BM = 8
_K = 6
_N = _ROWS // _BM
_ROUNDS = _N // _K


def _relu_stream(x_hbm, o_hbm, in_buf, out_buf, in_sem, out_sem):
    def in_copy(i, s):
        return pltpu.make_async_copy(
            x_hbm.at[pl.ds(i * _BM, _BM), :], in_buf.at[s], in_sem.at[s]
        )

    def out_copy(i, s):
        return pltpu.make_async_copy(
            out_buf.at[s], o_hbm.at[pl.ds(i * _BM, _BM), :], out_sem.at[s]
        )

    for s in range(_K):
        in_copy(s, s).start(priority=s % 2)

    def round_body(r, _):
        base = r * _K
        for s in range(_K):
            i = base + s
            in_copy(i, s).wait()

            @pl.when(r > 0)
            def _():
                out_copy(i - _K, s).wait()

            out_buf[s] = jnp.maximum(in_buf[s], 0.0)
            out_copy(i, s).start(priority=s % 2)

            @pl.when(r + 1 < _ROUNDS)
            def _():
                in_copy(i + _K, s).start(priority=s % 2)

        return ()

    jax.lax.fori_loop(0, _ROUNDS, round_body, (), unroll=False)

    for s in range(_K):
        out_copy((_ROUNDS - 1) * _K + s, s).wait()


def kernel(x):
    x2 = x.reshape(_ROWS, _COLS)
    out = pl.pallas_call(
        _relu_stream,
        in_specs=[pl.BlockSpec(memory_space=pl.ANY)],
        out_specs=pl.BlockSpec(memory_space=pl.ANY),
        out_shape=jax.ShapeDtypeStruct((_ROWS, _COLS), x.dtype),
        scratch_shapes=[
            pltpu.VMEM((_K, _BM, _COLS), jnp.float32),
            pltpu.VMEM((_K, _BM, _COLS), jnp.float32),
            pltpu.SemaphoreType.DMA((_K,)),
            pltpu.SemaphoreType.DMA((_K,)),
        ],
    )(x2)
    return out.reshape(x.shape)
